# Initial kernel scaffold; baseline (speedup 1.0000x reference)
import jax
import jax.numpy as jnp
from jax.experimental import pallas as pl


def kernel(hidden, W_out, b_out, boxes):
    raise NotImplementedError("write your pallas kernel here")


# speedup vs baseline: 11.1672x; 11.1672x over previous
import jax
import jax.numpy as jnp
from jax.experimental import pallas as pl
from jax.experimental.pallas import tpu as pltpu

_N, _H, _C = 5000, 512, 151
_NP = 5120
_CP = 152
_BN = 512


def _proj_kernel(h_ref, w_ref, brow_ref, bcol_ref, dists_ref, pt_ref):
    h = h_ref[...]
    w = w_ref[...]

    d_row = jax.lax.dot_general(h, w, (((1,), (1,)), ((), ())),
                                preferred_element_type=jnp.float32)
    dists_ref[...] = d_row + brow_ref[...]

    d_col = jax.lax.dot_general(w, h, (((1,), (1,)), ((), ())),
                                preferred_element_type=jnp.float32)
    d_col = d_col + bcol_ref[...]

    mx = jnp.max(d_col, axis=0, keepdims=True)
    e = jnp.exp(d_col - mx)
    p = e / jnp.sum(e, axis=0, keepdims=True)
    row_io = jax.lax.broadcasted_iota(jnp.int32, (_C, _BN), 0)
    p = jnp.where(row_io == 0, 0.0, p)

    g = pl.program_id(0) * _BN + jax.lax.broadcasted_iota(jnp.int32, (_C, _BN), 1)
    p = jnp.where(g < _N, p, -2.0)

    pt_ref[...] = jnp.concatenate(
        [p, jnp.full((_CP - _C, _BN), -2.0, jnp.float32)], axis=0)


def _decode_kernel(pt_ref, bx_ref, commit_ref, ps_ref):
    ps_ref[...] = pt_ref[...]
    commit_ref[...] = jnp.zeros((8, _NP), jnp.int32)

    lane1 = jax.lax.broadcasted_iota(jnp.int32, (1, _NP), 1)
    lane2 = jax.lax.broadcasted_iota(jnp.int32, (_CP, _NP), 1)
    row2 = jax.lax.broadcasted_iota(jnp.int32, (_CP, _NP), 0)
    flat = lane2 * _C + row2
    big = jnp.int32(2**31 - 1)

    x1 = bx_ref[0:1, :]
    y1 = bx_ref[1:2, :]
    x2 = bx_ref[2:3, :]
    y2 = bx_ref[3:4, :]
    area = (x2 - x1) * (y2 - y1)

    def body(it, carry):
        m_all = ps_ref[...]
        m = jnp.max(m_all)
        fmin = jnp.min(jnp.where(m_all == m, flat, big))
        i = fmin // _C
        c = fmin % _C

        onei = lane1 == i
        commit_ref[0:1, :] = jnp.where(onei, c, commit_ref[0:1, :])

        fonei = onei.astype(jnp.float32)
        xi1 = jnp.sum(fonei * x1)
        yi1 = jnp.sum(fonei * y1)
        xi2 = jnp.sum(fonei * x2)
        yi2 = jnp.sum(fonei * y2)
        ai = (xi2 - xi1) * (yi2 - yi1)
        iw = jnp.clip(jnp.minimum(x2, xi2) - jnp.maximum(x1, xi1), 0.0, None)
        ih = jnp.clip(jnp.minimum(y2, yi2) - jnp.maximum(y1, yi1), 0.0, None)
        inter = iw * ih
        union = jnp.maximum(area + ai - inter, 1e-8)
        ov = (inter / union) >= 0.5

        rowc = ps_ref[pl.ds(c, 1), :]
        ps_ref[pl.ds(c, 1), :] = jnp.where(ov, 0.0, rowc)
        ps_ref[...] = jnp.where(lane2 == i, -1.0, ps_ref[...])
        return carry

    jax.lax.fori_loop(0, _N, body, 0)


def kernel(hidden, W_out, b_out, boxes):
    hidden_p = jnp.zeros((_NP, _H), jnp.float32).at[:_N].set(hidden)
    brow = b_out.reshape(1, _C)
    bcol = b_out.reshape(_C, 1)

    dists_p, probs_t = pl.pallas_call(
        _proj_kernel,
        grid=(_NP // _BN,),
        in_specs=[
            pl.BlockSpec((_BN, _H), lambda j: (j, 0)),
            pl.BlockSpec((_C, _H), lambda j: (0, 0)),
            pl.BlockSpec((1, _C), lambda j: (0, 0)),
            pl.BlockSpec((_C, 1), lambda j: (0, 0)),
        ],
        out_specs=[
            pl.BlockSpec((_BN, _C), lambda j: (j, 0)),
            pl.BlockSpec((_CP, _BN), lambda j: (0, j)),
        ],
        out_shape=[
            jax.ShapeDtypeStruct((_NP, _C), jnp.float32),
            jax.ShapeDtypeStruct((_CP, _NP), jnp.float32),
        ],
    )(hidden_p, W_out, brow, bcol)

    bx = jnp.zeros((8, _NP), jnp.float32).at[:4, :_N].set(boxes.T)

    commit = pl.pallas_call(
        _decode_kernel,
        out_shape=jax.ShapeDtypeStruct((8, _NP), jnp.int32),
        scratch_shapes=[pltpu.VMEM((_CP, _NP), jnp.float32)],
    )(probs_t, bx)

    return dists_p[:_N], commit[0, :_N]


# speedup vs baseline: 28.7941x; 2.5785x over previous
import jax
import jax.numpy as jnp
from jax.experimental import pallas as pl
from jax.experimental.pallas import tpu as pltpu

_N, _H, _C = 5000, 512, 151
_NP = 5120
_CP = 152
_BN = 512


def _proj_kernel(h_ref, w_ref, brow_ref, bcol_ref, dists_ref, pt_ref):
    h = h_ref[...]
    w = w_ref[...]

    d_row = jax.lax.dot_general(h, w, (((1,), (1,)), ((), ())),
                                preferred_element_type=jnp.float32)
    dists_ref[...] = d_row + brow_ref[...]

    d_col = jax.lax.dot_general(w, h, (((1,), (1,)), ((), ())),
                                preferred_element_type=jnp.float32)
    d_col = d_col + bcol_ref[...]

    mx = jnp.max(d_col, axis=0, keepdims=True)
    e = jnp.exp(d_col - mx)
    p = e / jnp.sum(e, axis=0, keepdims=True)
    row_io = jax.lax.broadcasted_iota(jnp.int32, (_C, _BN), 0)
    p = jnp.where(row_io == 0, 0.0, p)

    g = pl.program_id(0) * _BN + jax.lax.broadcasted_iota(jnp.int32, (_C, _BN), 1)
    p = jnp.where(g < _N, p, -2.0)

    pt_ref[...] = jnp.concatenate(
        [p, jnp.full((_CP - _C, _BN), -2.0, jnp.float32)], axis=0)


def _decode_kernel(pt_ref, bx_ref, commit_ref, ps_ref, cmax_ref, ccls_ref):
    ps_ref[...] = pt_ref[...]
    commit_ref[...] = jnp.zeros((8, _NP), jnp.int32)

    lane1 = jax.lax.broadcasted_iota(jnp.int32, (1, _NP), 1)
    row2 = jax.lax.broadcasted_iota(jnp.int32, (_CP, _NP), 0)
    big = jnp.int32(2**31 - 1)

    m0 = pt_ref[...]
    cm = jnp.max(m0, axis=0, keepdims=True)
    cmax_ref[...] = cm
    ccls_ref[...] = jnp.min(jnp.where(m0 == cm, row2, big), axis=0,
                            keepdims=True)

    x1 = bx_ref[0:1, :]
    y1 = bx_ref[1:2, :]
    x2 = bx_ref[2:3, :]
    y2 = bx_ref[3:4, :]
    area = (x2 - x1) * (y2 - y1)

    def body(it, carry):
        cmax = cmax_ref[...]
        ccls = ccls_ref[...]
        m = jnp.max(cmax)
        atmax = cmax == m
        i = jnp.min(jnp.where(atmax, lane1, big))
        onei = lane1 == i
        c = jnp.sum(jnp.where(onei, ccls, 0))

        commit_ref[0:1, :] = jnp.where(onei, c, commit_ref[0:1, :])

        fonei = onei.astype(jnp.float32)
        xi1 = jnp.sum(fonei * x1)
        yi1 = jnp.sum(fonei * y1)
        xi2 = jnp.sum(fonei * x2)
        yi2 = jnp.sum(fonei * y2)
        ai = (xi2 - xi1) * (yi2 - yi1)
        iw = jnp.clip(jnp.minimum(x2, xi2) - jnp.maximum(x1, xi1), 0.0, None)
        ih = jnp.clip(jnp.minimum(y2, yi2) - jnp.maximum(y1, yi1), 0.0, None)
        inter = iw * ih
        union = jnp.maximum(area + ai - inter, 1e-8)
        ov = (inter / union) >= 0.5

        rowc = ps_ref[pl.ds(c, 1), :]
        ps_ref[pl.ds(c, 1), :] = jnp.where(ov, 0.0, rowc)

        cmax = jnp.where(onei, -1.0, cmax)
        cmax_ref[...] = cmax

        aff = ov & (ccls == c) & (cmax >= 0.0)

        @pl.when(jnp.any(aff))
        def _rescan():
            mm = ps_ref[...]
            nm = jnp.max(mm, axis=0, keepdims=True)
            ncls = jnp.min(jnp.where(mm == nm, row2, big), axis=0,
                           keepdims=True)
            cmax_ref[...] = jnp.where(aff, nm, cmax)
            ccls_ref[...] = jnp.where(aff, ncls, ccls)

        return carry

    jax.lax.fori_loop(0, _N, body, 0)


def kernel(hidden, W_out, b_out, boxes):
    hidden_p = jnp.zeros((_NP, _H), jnp.float32).at[:_N].set(hidden)
    brow = b_out.reshape(1, _C)
    bcol = b_out.reshape(_C, 1)

    dists_p, probs_t = pl.pallas_call(
        _proj_kernel,
        grid=(_NP // _BN,),
        in_specs=[
            pl.BlockSpec((_BN, _H), lambda j: (j, 0)),
            pl.BlockSpec((_C, _H), lambda j: (0, 0)),
            pl.BlockSpec((1, _C), lambda j: (0, 0)),
            pl.BlockSpec((_C, 1), lambda j: (0, 0)),
        ],
        out_specs=[
            pl.BlockSpec((_BN, _C), lambda j: (j, 0)),
            pl.BlockSpec((_CP, _BN), lambda j: (0, j)),
        ],
        out_shape=[
            jax.ShapeDtypeStruct((_NP, _C), jnp.float32),
            jax.ShapeDtypeStruct((_CP, _NP), jnp.float32),
        ],
    )(hidden_p, W_out, brow, bcol)

    bx = jnp.zeros((8, _NP), jnp.float32).at[:4, :_N].set(boxes.T)

    commit = pl.pallas_call(
        _decode_kernel,
        out_shape=jax.ShapeDtypeStruct((8, _NP), jnp.int32),
        scratch_shapes=[
            pltpu.VMEM((_CP, _NP), jnp.float32),
            pltpu.VMEM((1, _NP), jnp.float32),
            pltpu.VMEM((1, _NP), jnp.int32),
        ],
    )(probs_t, bx)

    return dists_p[:_N], commit[0, :_N]


# speedup vs baseline: 40.8385x; 1.4183x over previous
import jax
import jax.numpy as jnp
from jax.experimental import pallas as pl
from jax.experimental.pallas import tpu as pltpu

_N, _H, _C = 5000, 512, 151
_NP = 5120
_CP = 152
_BN = 512


def _proj_kernel(h_ref, w_ref, brow_ref, bcol_ref, dists_ref, pt_ref):
    h = h_ref[...]
    w = w_ref[...]

    d_row = jax.lax.dot_general(h, w, (((1,), (1,)), ((), ())),
                                preferred_element_type=jnp.float32)
    dists_ref[...] = d_row + brow_ref[...]

    d_col = jax.lax.dot_general(w, h, (((1,), (1,)), ((), ())),
                                preferred_element_type=jnp.float32)
    d_col = d_col + bcol_ref[...]

    mx = jnp.max(d_col, axis=0, keepdims=True)
    e = jnp.exp(d_col - mx)
    p = e / jnp.sum(e, axis=0, keepdims=True)
    row_io = jax.lax.broadcasted_iota(jnp.int32, (_C, _BN), 0)
    p = jnp.where(row_io == 0, 0.0, p)

    g = pl.program_id(0) * _BN + jax.lax.broadcasted_iota(jnp.int32, (_C, _BN), 1)
    p = jnp.where(g < _N, p, -2.0)

    pt_ref[...] = jnp.concatenate(
        [p, jnp.full((_CP - _C, _BN), -2.0, jnp.float32)], axis=0)


_NB = _NP // 128


def _decode_kernel(pt_ref, x1_ref, y1_ref, x2_ref, y2_ref,
                   commit_ref, ps_ref, cmax_ref, ccls_ref):
    ps_ref[...] = pt_ref[...]
    commit_ref[...] = jnp.zeros((_NB, 128), jnp.int32)

    r_io = jax.lax.broadcasted_iota(jnp.int32, (_NB, 128), 0)
    l_io = jax.lax.broadcasted_iota(jnp.int32, (_NB, 128), 1)
    bidx = r_io * 128 + l_io
    row3 = jax.lax.broadcasted_iota(jnp.int32, (_CP, _NB, 128), 0)
    big = jnp.int32(2**31 - 1)

    m0 = pt_ref[...]
    cm = jnp.max(m0, axis=0)
    cmax_ref[...] = cm
    ccls_ref[...] = jnp.min(jnp.where(m0 == cm[None], row3, big), axis=0)

    x1 = x1_ref[...]
    y1 = y1_ref[...]
    x2 = x2_ref[...]
    y2 = y2_ref[...]
    area = (x2 - x1) * (y2 - y1)

    def body(it, carry):
        cmax = cmax_ref[...]
        ccls = ccls_ref[...]
        m = jnp.max(cmax)
        atmax = cmax == m
        i = jnp.min(jnp.where(atmax, bidx, big))
        onei = bidx == i
        c = jnp.sum(jnp.where(onei, ccls, 0))

        commit_ref[...] = jnp.where(onei, c, commit_ref[...])

        fonei = onei.astype(jnp.float32)
        xi1 = jnp.sum(fonei * x1)
        yi1 = jnp.sum(fonei * y1)
        xi2 = jnp.sum(fonei * x2)
        yi2 = jnp.sum(fonei * y2)
        ai = (xi2 - xi1) * (yi2 - yi1)
        iw = jnp.clip(jnp.minimum(x2, xi2) - jnp.maximum(x1, xi1), 0.0, None)
        ih = jnp.clip(jnp.minimum(y2, yi2) - jnp.maximum(y1, yi1), 0.0, None)
        inter = iw * ih
        union = jnp.maximum(area + ai - inter, 1e-8)
        ov = (inter / union) >= 0.5

        rowc = ps_ref[pl.ds(c, 1), :, :]
        ps_ref[pl.ds(c, 1), :, :] = jnp.where(ov[None], 0.0, rowc)

        cmax = jnp.where(onei, -1.0, cmax)
        cmax_ref[...] = cmax

        aff = ov & (ccls == c) & (cmax >= 0.0)

        @pl.when(jnp.any(aff))
        def _rescan():
            mm = ps_ref[...]
            nm = jnp.max(mm, axis=0)
            ncls = jnp.min(jnp.where(mm == nm[None], row3, big), axis=0)
            cmax_ref[...] = jnp.where(aff, nm, cmax)
            ccls_ref[...] = jnp.where(aff, ncls, ccls)

        return carry

    jax.lax.fori_loop(0, _N, body, 0)


def kernel(hidden, W_out, b_out, boxes):
    hidden_p = jnp.zeros((_NP, _H), jnp.float32).at[:_N].set(hidden)
    brow = b_out.reshape(1, _C)
    bcol = b_out.reshape(_C, 1)

    dists_p, probs_t = pl.pallas_call(
        _proj_kernel,
        grid=(_NP // _BN,),
        in_specs=[
            pl.BlockSpec((_BN, _H), lambda j: (j, 0)),
            pl.BlockSpec((_C, _H), lambda j: (0, 0)),
            pl.BlockSpec((1, _C), lambda j: (0, 0)),
            pl.BlockSpec((_C, 1), lambda j: (0, 0)),
        ],
        out_specs=[
            pl.BlockSpec((_BN, _C), lambda j: (j, 0)),
            pl.BlockSpec((_CP, _BN), lambda j: (0, j)),
        ],
        out_shape=[
            jax.ShapeDtypeStruct((_NP, _C), jnp.float32),
            jax.ShapeDtypeStruct((_CP, _NP), jnp.float32),
        ],
    )(hidden_p, W_out, brow, bcol)

    probs3 = probs_t.reshape(_CP, _NB, 128)
    bcols = [
        jnp.zeros((_NP,), jnp.float32).at[:_N].set(boxes[:, k]).reshape(_NB, 128)
        for k in range(4)
    ]

    commit = pl.pallas_call(
        _decode_kernel,
        out_shape=jax.ShapeDtypeStruct((_NB, 128), jnp.int32),
        scratch_shapes=[
            pltpu.VMEM((_CP, _NB, 128), jnp.float32),
            pltpu.VMEM((_NB, 128), jnp.float32),
            pltpu.VMEM((_NB, 128), jnp.int32),
        ],
    )(probs3, *bcols)

    return dists_p[:_N], commit.reshape(_NP)[:_N]


# speedup vs baseline: 42.2250x; 1.0339x over previous
import jax
import jax.numpy as jnp
from jax.experimental import pallas as pl
from jax.experimental.pallas import tpu as pltpu

_N, _H, _C = 5000, 512, 151
_NP = 5120
_CP = 152
_BN = 512


def _proj_kernel(h_ref, w_ref, brow_ref, bcol_ref, dists_ref, pt_ref):
    h = h_ref[...]
    w = w_ref[...]

    d_row = jax.lax.dot_general(h, w, (((1,), (1,)), ((), ())),
                                preferred_element_type=jnp.float32)
    dists_ref[...] = d_row + brow_ref[...]

    d_col = jax.lax.dot_general(w, h, (((1,), (1,)), ((), ())),
                                preferred_element_type=jnp.float32)
    d_col = d_col + bcol_ref[...]

    mx = jnp.max(d_col, axis=0, keepdims=True)
    e = jnp.exp(d_col - mx)
    p = e / jnp.sum(e, axis=0, keepdims=True)
    row_io = jax.lax.broadcasted_iota(jnp.int32, (_C, _BN), 0)
    p = jnp.where(row_io == 0, 0.0, p)

    g = pl.program_id(0) * _BN + jax.lax.broadcasted_iota(jnp.int32, (_C, _BN), 1)
    p = jnp.where(g < _N, p, -2.0)

    pt_ref[...] = jnp.concatenate(
        [p, jnp.full((_CP - _C, _BN), -2.0, jnp.float32)], axis=0)


_NB = _NP // 128


def _decode_kernel(pt_ref, x1_ref, y1_ref, x2_ref, y2_ref, bxr_ref,
                   commit_ref, ps_ref, cmax_ref, tag_ref):
    ps_ref[...] = pt_ref[...]
    commit_ref[...] = jnp.zeros((_NB, 128), jnp.int32)

    r_io = jax.lax.broadcasted_iota(jnp.int32, (_NB, 128), 0)
    l_io = jax.lax.broadcasted_iota(jnp.int32, (_NB, 128), 1)
    bidx = r_io * 128 + l_io
    row3 = jax.lax.broadcasted_iota(jnp.int32, (_CP, _NB, 128), 0)
    big = jnp.int32(2**31 - 1)

    m0 = pt_ref[...]
    cm = jnp.max(m0, axis=0)
    cmax_ref[...] = cm
    tag_ref[...] = bidx * 256 + jnp.min(
        jnp.where(m0 == cm[None], row3, big), axis=0)

    x1 = x1_ref[...]
    y1 = y1_ref[...]
    x2 = x2_ref[...]
    y2 = y2_ref[...]
    area = (x2 - x1) * (y2 - y1)

    def body(it, carry):
        cmax = cmax_ref[...]
        tag = tag_ref[...]
        m = jnp.max(cmax)
        atmax = cmax == m
        fmin = jnp.min(jnp.where(atmax, tag, big))
        i = fmin // 256
        c = fmin % 256
        onei = bidx == i

        commit_ref[...] = jnp.where(onei, c, commit_ref[...])

        xi1 = bxr_ref[i, 0]
        yi1 = bxr_ref[i, 1]
        xi2 = bxr_ref[i, 2]
        yi2 = bxr_ref[i, 3]
        ai = (xi2 - xi1) * (yi2 - yi1)
        iw = jnp.clip(jnp.minimum(x2, xi2) - jnp.maximum(x1, xi1), 0.0, None)
        ih = jnp.clip(jnp.minimum(y2, yi2) - jnp.maximum(y1, yi1), 0.0, None)
        inter = iw * ih
        union = jnp.maximum(area + ai - inter, 1e-8)
        ov = (inter / union) >= 0.5

        rowc = ps_ref[pl.ds(c, 1), :, :]
        ps_ref[pl.ds(c, 1), :, :] = jnp.where(ov[None], 0.0, rowc)

        cmax = jnp.where(onei, -1.0, cmax)
        cmax_ref[...] = cmax

        aff = ov & (tag % 256 == c) & (cmax >= 0.0)

        @pl.when(jnp.any(aff))
        def _rescan():
            mm = ps_ref[...]
            nm = jnp.max(mm, axis=0)
            ncls = jnp.min(jnp.where(mm == nm[None], row3, big), axis=0)
            cmax_ref[...] = jnp.where(aff, nm, cmax)
            tag_ref[...] = jnp.where(aff, bidx * 256 + ncls, tag)

        return carry

    jax.lax.fori_loop(0, _N, body, 0)


def kernel(hidden, W_out, b_out, boxes):
    hidden_p = jnp.zeros((_NP, _H), jnp.float32).at[:_N].set(hidden)
    brow = b_out.reshape(1, _C)
    bcol = b_out.reshape(_C, 1)

    dists_p, probs_t = pl.pallas_call(
        _proj_kernel,
        grid=(_NP // _BN,),
        in_specs=[
            pl.BlockSpec((_BN, _H), lambda j: (j, 0)),
            pl.BlockSpec((_C, _H), lambda j: (0, 0)),
            pl.BlockSpec((1, _C), lambda j: (0, 0)),
            pl.BlockSpec((_C, 1), lambda j: (0, 0)),
        ],
        out_specs=[
            pl.BlockSpec((_BN, _C), lambda j: (j, 0)),
            pl.BlockSpec((_CP, _BN), lambda j: (0, j)),
        ],
        out_shape=[
            jax.ShapeDtypeStruct((_NP, _C), jnp.float32),
            jax.ShapeDtypeStruct((_CP, _NP), jnp.float32),
        ],
    )(hidden_p, W_out, brow, bcol)

    probs3 = probs_t.reshape(_CP, _NB, 128)
    bcols = [
        jnp.zeros((_NP,), jnp.float32).at[:_N].set(boxes[:, k]).reshape(_NB, 128)
        for k in range(4)
    ]
    bxr = jnp.zeros((_NP, 8), jnp.float32).at[:_N, :4].set(boxes)

    commit = pl.pallas_call(
        _decode_kernel,
        out_shape=jax.ShapeDtypeStruct((_NB, 128), jnp.int32),
        scratch_shapes=[
            pltpu.VMEM((_CP, _NB, 128), jnp.float32),
            pltpu.VMEM((_NB, 128), jnp.float32),
            pltpu.VMEM((_NB, 128), jnp.int32),
        ],
    )(probs3, *bcols, bxr)

    return dists_p[:_N], commit.reshape(_NP)[:_N]
